# baseline (speedup 1.0000x reference)
import functools

import jax
import jax.numpy as jnp
from jax import lax
from jax.experimental import pallas as pl
from jax.experimental.pallas import tpu as pltpu
from jax.experimental.pallas import tpu_sc as plsc

CH = 128
EP = 128


def _build(n_cores, n_workers, n_chunks, vocab, maxlen):
    per_w = n_chunks * CH
    total = n_workers * per_w
    mesh = plsc.VectorSubcoreMesh(core_axis_name="c", subcore_axis_name="s")
    nbuf = 4
    npat = 25
    n_steps = -(-(n_chunks + 2) // nbuf)

    @functools.partial(
        pl.kernel,
        out_type=jax.ShapeDtypeStruct((total, EP), jnp.float32),
        mesh=mesh,
        scratch_types=[
            pltpu.VMEM((n_chunks, CH), jnp.int32),
            pltpu.VMEM((npat, CH), jnp.int32),
            pltpu.VMEM((nbuf, CH, EP), jnp.float32),
        ]
        + [pltpu.SemaphoreType.DMA] * (3 * nbuf),
        compiler_params=pltpu.CompilerParams(use_tc_tiling_on_sc=False),
    )
    def kern(x_hbm, tok_hbm, pos_hbm, pidx_hbm, out_hbm, idx_v, pidx_v, rows, *sems):
        gsem = sems[0:nbuf]
        asem = sems[nbuf:2 * nbuf]
        osem = sems[2 * nbuf:3 * nbuf]
        wid = lax.axis_index("s") * n_cores + lax.axis_index("c")
        base = wid * per_w

        pltpu.sync_copy(x_hbm.at[wid], idx_v)
        pltpu.sync_copy(pidx_hbm, pidx_v)

        def step(t0, carry):
            for k in range(nbuf):
                t = t0 * nbuf + k

                @pl.when(jnp.logical_and(t >= nbuf, t < n_chunks))
                def _():
                    pltpu.make_async_copy(
                        rows.at[k], out_hbm.at[pl.ds(0, CH)], osem[k]).wait()

                @pl.when(t < n_chunks)
                def _():
                    pltpu.async_copy(tok_hbm.at[idx_v.at[t]], rows.at[k], gsem[k])

                c1 = t - 1
                b1 = (k - 1) % nbuf

                @pl.when(jnp.logical_and(c1 >= 0, c1 < n_chunks))
                def _():
                    pltpu.make_async_copy(
                        tok_hbm.at[idx_v.at[c1]], rows.at[b1], gsem[b1]).wait()
                    pltpu.async_copy(
                        pos_hbm.at[pidx_v.at[c1 % npat]], rows.at[b1],
                        asem[b1], add=True)

                c2 = t - 2
                b2 = (k - 2) % nbuf

                @pl.when(jnp.logical_and(c2 >= 0, c2 < n_chunks))
                def _():
                    pltpu.make_async_copy(
                        pos_hbm.at[pidx_v.at[c2 % npat]], rows.at[b2],
                        asem[b2]).wait()
                    pltpu.async_copy(
                        rows.at[b2], out_hbm.at[pl.ds(base + c2 * CH, CH)], osem[b2])

            return carry

        lax.fori_loop(0, n_steps, step, 0)

        for b in range(nbuf):
            pltpu.make_async_copy(
                rows.at[b], out_hbm.at[pl.ds(0, CH)], osem[b]).wait()

    return kern


def kernel(x, token_table, pos_table):
    batch, maxlen = x.shape
    vocab, embed = token_table.shape
    info = plsc.get_sparse_core_info()
    n_workers = info.num_cores * info.num_subcores
    total = batch * maxlen
    per_w = total // n_workers
    assert total % n_workers == 0 and per_w % CH == 0 and per_w % maxlen == 0
    n_chunks = per_w // CH

    tok128 = jnp.pad(token_table, ((0, 0), (0, EP - embed)))
    pos128 = jnp.pad(pos_table, ((0, 0), (0, EP - embed)))
    xr = x.reshape(n_workers, n_chunks, CH).astype(jnp.int32)
    pidx = (jnp.arange(25 * CH, dtype=jnp.int32) % maxlen).reshape(25, CH)
    kern = _build(info.num_cores, n_workers, n_chunks, vocab, maxlen)
    out = kern(xr, tok128, pos128, pidx)
    return out[:, :embed].reshape(batch, maxlen, embed)


# speedup vs baseline: 1.3895x; 1.3895x over previous
import functools

import jax
import jax.numpy as jnp
from jax import lax
from jax.experimental import pallas as pl
from jax.experimental.pallas import tpu as pltpu
from jax.experimental.pallas import tpu_sc as plsc

CH = 128
EP = 128


def _build(n_cores, n_workers, n_chunks, embed, maxlen):
    per_w = n_chunks * CH
    total = n_workers * per_w
    mesh = plsc.VectorSubcoreMesh(core_axis_name="c", subcore_axis_name="s")
    nbuf = 4
    n_steps = -(-(n_chunks + 2) // nbuf)
    nvec = embed // 16

    @functools.partial(
        pl.kernel,
        out_type=jax.ShapeDtypeStruct((total, EP), jnp.float32),
        mesh=mesh,
        scratch_types=[
            pltpu.VMEM((n_chunks, CH), jnp.int32),
            pltpu.VMEM((maxlen, embed), jnp.float32),
            pltpu.VMEM((nbuf, CH, EP), jnp.float32),
        ]
        + [pltpu.SemaphoreType.DMA] * (2 * nbuf),
        compiler_params=pltpu.CompilerParams(use_tc_tiling_on_sc=False),
    )
    def kern(x_hbm, tok_hbm, pos_hbm, out_hbm, idx_v, pos_v, rows, *sems):
        gsem = sems[0:nbuf]
        osem = sems[nbuf:2 * nbuf]
        wid = lax.axis_index("s") * n_cores + lax.axis_index("c")
        base = wid * per_w

        pltpu.sync_copy(x_hbm.at[wid], idx_v)
        pltpu.sync_copy(pos_hbm, pos_v)

        def step(t0, carry):
            for k in range(nbuf):
                t = t0 * nbuf + k

                @pl.when(jnp.logical_and(t >= nbuf, t < n_chunks))
                def _():
                    pltpu.make_async_copy(
                        rows.at[k], out_hbm.at[pl.ds(0, CH)], osem[k]).wait()

                @pl.when(t < n_chunks)
                def _():
                    pltpu.async_copy(tok_hbm.at[idx_v.at[t]], rows.at[k], gsem[k])

                c2 = t - 2
                b2 = (k - 2) % nbuf

                @pl.when(jnp.logical_and(c2 >= 0, c2 < n_chunks))
                def _():
                    pltpu.make_async_copy(
                        tok_hbm.at[idx_v.at[c2]], rows.at[b2], gsem[b2]).wait()
                    pbase = lax.rem(c2 * CH, maxlen)

                    def add_row(j, _):
                        p = pbase + j
                        p = jnp.where(p >= maxlen, p - maxlen, p)
                        for v in range(nvec):
                            sl = pl.ds(v * 16, 16)
                            rows[b2, j, sl] = rows[b2, j, sl] + pos_v[p, sl]
                        return _

                    lax.fori_loop(0, CH, add_row, 0)
                    pltpu.async_copy(
                        rows.at[b2], out_hbm.at[pl.ds(base + c2 * CH, CH)], osem[b2])

            return carry

        lax.fori_loop(0, n_steps, step, 0)

        for b in range(nbuf):
            pltpu.make_async_copy(
                rows.at[b], out_hbm.at[pl.ds(0, CH)], osem[b]).wait()

    return kern


def kernel(x, token_table, pos_table):
    batch, maxlen = x.shape
    vocab, embed = token_table.shape
    info = plsc.get_sparse_core_info()
    n_workers = info.num_cores * info.num_subcores
    total = batch * maxlen
    per_w = total // n_workers
    assert total % n_workers == 0 and per_w % CH == 0 and per_w % maxlen == 0
    n_chunks = per_w // CH

    tok128 = jnp.pad(token_table, ((0, 0), (0, EP - embed)))
    xr = x.reshape(n_workers, n_chunks, CH).astype(jnp.int32)
    kern = _build(info.num_cores, n_workers, n_chunks, embed, maxlen)
    out = kern(xr, tok128, pos_table)
    return out[:, :embed].reshape(batch, maxlen, embed)


# speedup vs baseline: 1.5204x; 1.0942x over previous
import functools

import jax
import jax.numpy as jnp
from jax import lax
from jax.experimental import pallas as pl
from jax.experimental.pallas import tpu as pltpu
from jax.experimental.pallas import tpu_sc as plsc

CH = 128
EP = 128


def _build(n_cores, n_workers, n_chunks, embed, maxlen):
    per_w = n_chunks * CH
    total = n_workers * per_w
    mesh = plsc.VectorSubcoreMesh(core_axis_name="c", subcore_axis_name="s")
    nbuf = 4
    n_steps = -(-(n_chunks + 2) // nbuf)
    nvec = embed // 16

    @functools.partial(
        pl.kernel,
        out_type=jax.ShapeDtypeStruct((total, EP), jnp.float32),
        mesh=mesh,
        scratch_types=[
            pltpu.VMEM((n_chunks, CH), jnp.int32),
            pltpu.VMEM((maxlen, embed), jnp.float32),
            pltpu.VMEM((nbuf, CH, EP), jnp.float32),
        ]
        + [pltpu.SemaphoreType.DMA] * (2 * nbuf),
        compiler_params=pltpu.CompilerParams(use_tc_tiling_on_sc=False),
    )
    def kern(x_hbm, tok_hbm, pos_hbm, out_hbm, idx_v, pos_v, rows, *sems):
        gsem = sems[0:nbuf]
        osem = sems[nbuf:2 * nbuf]
        wid = lax.axis_index("s") * n_cores + lax.axis_index("c")
        base = wid * per_w

        pltpu.sync_copy(x_hbm.at[wid], idx_v)
        pltpu.sync_copy(pos_hbm, pos_v)

        def step(t0, carry):
            for k in range(nbuf):
                t = t0 * nbuf + k

                @pl.when(jnp.logical_and(t >= nbuf, t < n_chunks))
                def _():
                    pltpu.make_async_copy(
                        rows.at[k], out_hbm.at[pl.ds(0, CH)], osem[k]).wait()

                @pl.when(t < n_chunks)
                def _():
                    pltpu.async_copy(tok_hbm.at[idx_v.at[t]], rows.at[k], gsem[k])

                c2 = t - 2
                b2 = (k - 2) % nbuf

                @pl.when(jnp.logical_and(c2 >= 0, c2 < n_chunks))
                def _():
                    pltpu.make_async_copy(
                        tok_hbm.at[idx_v.at[c2]], rows.at[b2], gsem[b2]).wait()
                    pbase = lax.rem(c2 * CH, maxlen)
                    m = jnp.minimum(maxlen - pbase, CH)

                    def add_rows(off):
                        def body(j, _):
                            p = j + off
                            for v in range(nvec):
                                sl = pl.ds(v * 16, 16)
                                plsc.addupdate(rows.at[b2, j, sl], pos_v[p, sl])
                            return _
                        return body

                    lax.fori_loop(0, m, add_rows(pbase), 0)
                    lax.fori_loop(m, CH, add_rows(pbase - maxlen), 0)
                    pltpu.async_copy(
                        rows.at[b2], out_hbm.at[pl.ds(base + c2 * CH, CH)], osem[b2])

            return carry

        lax.fori_loop(0, n_steps, step, 0)

        for b in range(nbuf):
            pltpu.make_async_copy(
                rows.at[b], out_hbm.at[pl.ds(0, CH)], osem[b]).wait()

    return kern


def kernel(x, token_table, pos_table):
    batch, maxlen = x.shape
    vocab, embed = token_table.shape
    info = plsc.get_sparse_core_info()
    n_workers = info.num_cores * info.num_subcores
    total = batch * maxlen
    per_w = total // n_workers
    assert total % n_workers == 0 and per_w % CH == 0 and per_w % maxlen == 0
    n_chunks = per_w // CH

    tok128 = jnp.pad(token_table, ((0, 0), (0, EP - embed)))
    xr = x.reshape(n_workers, n_chunks, CH).astype(jnp.int32)
    kern = _build(info.num_cores, n_workers, n_chunks, embed, maxlen)
    out = kern(xr, tok128, pos_table)
    return out[:, :embed].reshape(batch, maxlen, embed)


# speedup vs baseline: 1.5699x; 1.0326x over previous
import functools

import jax
import jax.numpy as jnp
from jax import lax
from jax.experimental import pallas as pl
from jax.experimental.pallas import tpu as pltpu
from jax.experimental.pallas import tpu_sc as plsc

CH = 128
EP = 128


def _build(n_cores, n_workers, n_chunks, embed, maxlen):
    per_w = n_chunks * CH
    total = n_workers * per_w
    mesh = plsc.VectorSubcoreMesh(core_axis_name="c", subcore_axis_name="s")
    nbuf = 4
    n_steps = -(-(n_chunks + 2) // nbuf)
    nvec = embed // 16

    @functools.partial(
        pl.kernel,
        out_type=jax.ShapeDtypeStruct((total, EP), jnp.float32),
        mesh=mesh,
        scratch_types=[
            pltpu.VMEM((n_chunks, CH), jnp.int32),
            pltpu.VMEM((maxlen, embed), jnp.float32),
            pltpu.VMEM((nbuf, CH, EP), jnp.float32),
        ]
        + [pltpu.SemaphoreType.DMA] * (2 * nbuf),
        compiler_params=pltpu.CompilerParams(use_tc_tiling_on_sc=False),
    )
    def kern(x_hbm, tok_hbm, pos_hbm, out_hbm, idx_v, pos_v, rows, *sems):
        gsem = sems[0:nbuf]
        osem = sems[nbuf:2 * nbuf]
        wid = lax.axis_index("s") * n_cores + lax.axis_index("c")
        base = wid * per_w

        pltpu.sync_copy(x_hbm.at[wid], idx_v)
        pltpu.sync_copy(pos_hbm, pos_v)

        def step(t0, carry):
            for k in range(nbuf):
                t = t0 * nbuf + k

                @pl.when(jnp.logical_and(t >= nbuf, t < n_chunks))
                def _():
                    pltpu.make_async_copy(
                        rows.at[k], out_hbm.at[pl.ds(0, CH)], osem[k]).wait()

                @pl.when(t < n_chunks)
                def _():
                    pltpu.async_copy(tok_hbm.at[idx_v.at[t]], rows.at[k], gsem[k])

                c2 = t - 2
                b2 = (k - 2) % nbuf

                @pl.when(jnp.logical_and(c2 >= 0, c2 < n_chunks))
                def _():
                    pltpu.make_async_copy(
                        tok_hbm.at[idx_v.at[c2]], rows.at[b2], gsem[b2]).wait()
                    pbase = lax.rem(c2 * CH, maxlen)

                    def add_rows(i, _):
                        for jj in range(4):
                            j = i * 4 + jj
                            p = pbase + j
                            p = jnp.where(p >= maxlen, p - maxlen, p)
                            for v in range(nvec):
                                sl = pl.ds(v * 16, 16)
                                plsc.addupdate(rows.at[b2, j, sl], pos_v[p, sl])
                        return _

                    lax.fori_loop(0, CH // 4, add_rows, 0)
                    pltpu.async_copy(
                        rows.at[b2], out_hbm.at[pl.ds(base + c2 * CH, CH)], osem[b2])

            return carry

        lax.fori_loop(0, n_steps, step, 0)

        for b in range(nbuf):
            pltpu.make_async_copy(
                rows.at[b], out_hbm.at[pl.ds(0, CH)], osem[b]).wait()

    return kern


def kernel(x, token_table, pos_table):
    batch, maxlen = x.shape
    vocab, embed = token_table.shape
    info = plsc.get_sparse_core_info()
    n_workers = info.num_cores * info.num_subcores
    total = batch * maxlen
    per_w = total // n_workers
    assert total % n_workers == 0 and per_w % CH == 0 and per_w % maxlen == 0
    n_chunks = per_w // CH

    tok128 = jnp.pad(token_table, ((0, 0), (0, EP - embed)))
    xr = x.reshape(n_workers, n_chunks, CH).astype(jnp.int32)
    kern = _build(info.num_cores, n_workers, n_chunks, embed, maxlen)
    out = kern(xr, tok128, pos_table)
    return out[:, :embed].reshape(batch, maxlen, embed)
